# baseline (speedup 1.0000x reference)
import functools

import jax
import jax.numpy as jnp
from jax import lax
from jax.experimental import pallas as pl
from jax.experimental.pallas import tpu as pltpu
from jax.experimental.pallas import tpu_sc as plsc

N = 10000
E = 320000
NFEAT = 128
NHID = 128
NCLASS = 64

NC = 2
NS = 16
LANES = 16

BR = 400


def _mm_dual_body(x_ref, y_ref, w_ref, ox_ref, oy_ref):
    ox_ref[...] = jnp.dot(x_ref[...], w_ref[...], preferred_element_type=jnp.float32)
    oy_ref[...] = jnp.dot(y_ref[...], w_ref[...], preferred_element_type=jnp.float32)


def _mm_dual(x, y, W1):
    n, k = x.shape
    h = W1.shape[1]
    return pl.pallas_call(
        _mm_dual_body,
        grid=(n // BR,),
        in_specs=[
            pl.BlockSpec((BR, k), lambda i: (i, 0)),
            pl.BlockSpec((BR, k), lambda i: (i, 0)),
            pl.BlockSpec((k, h), lambda i: (0, 0)),
        ],
        out_specs=[
            pl.BlockSpec((BR, h), lambda i: (i, 0)),
            pl.BlockSpec((BR, h), lambda i: (i, 0)),
        ],
        out_shape=[
            jax.ShapeDtypeStruct((n, h), jnp.float32),
            jax.ShapeDtypeStruct((n, h), jnp.float32),
        ],
    )(x, y, W1)


def _stage_b_body(tx_ref, ty_ref, b1_ref, w2_ref, ox_ref, oy_ref):
    hx = jnp.maximum(tx_ref[...] + b1_ref[...], 0.0)
    hy = jnp.maximum(ty_ref[...] + b1_ref[...], 0.0)
    ox_ref[...] = jnp.dot(hx, w2_ref[...], preferred_element_type=jnp.float32)
    oy_ref[...] = jnp.dot(hy, w2_ref[...], preferred_element_type=jnp.float32)


def _stage_b(t, b1, W2):
    n2, h = t.shape
    n = n2 // 2
    c = W2.shape[1]
    nb = n // BR
    return pl.pallas_call(
        _stage_b_body,
        grid=(nb,),
        in_specs=[
            pl.BlockSpec((BR, h), lambda i: (i, 0)),
            pl.BlockSpec((BR, h), lambda i: (i + nb, 0)),
            pl.BlockSpec((1, h), lambda i: (0, 0)),
            pl.BlockSpec((h, c), lambda i: (0, 0)),
        ],
        out_specs=[
            pl.BlockSpec((BR, c), lambda i: (i, 0)),
            pl.BlockSpec((BR, c), lambda i: (i, 0)),
        ],
        out_shape=[
            jax.ShapeDtypeStruct((n, c), jnp.float32),
            jax.ShapeDtypeStruct((n, c), jnp.float32),
        ],
    )(t, t, b1.reshape(1, h), W2)


def _stage_c_body(ux_ref, uy_ref, b2_ref, o_ref):
    zx = ux_ref[...] + b2_ref[...]
    zy = uy_ref[...] + b2_ref[...]
    z = jnp.concatenate([zx, zy], axis=1)
    m = jnp.max(z, axis=1, keepdims=True)
    ez = jnp.exp(z - m)
    lse = jnp.log(jnp.sum(ez, axis=1, keepdims=True)) + m
    o_ref[...] = z - lse


def _stage_c(u, b2):
    n2, c = u.shape
    n = n2 // 2
    nb = n // BR
    return pl.pallas_call(
        _stage_c_body,
        grid=(nb,),
        in_specs=[
            pl.BlockSpec((BR, c), lambda i: (i, 0)),
            pl.BlockSpec((BR, c), lambda i: (i + nb, 0)),
            pl.BlockSpec((1, c), lambda i: (0, 0)),
        ],
        out_specs=pl.BlockSpec((BR, 2 * c), lambda i: (i, 0)),
        out_shape=jax.ShapeDtypeStruct((n, 2 * c), jnp.float32),
    )(u, u, b2.reshape(1, c))


def _make_spmm(n, e, f, split, sc_tiling=False):
    half = split == "half"
    C = 64
    epw = e // NS if half else e // (NC * NS)
    nchunks = epw // C
    npairs = nchunks // 2
    NG = 5
    GC = nchunks // NG
    G = GC * C
    ppg = GC // 2
    RA = -(-n // NS) // 8 * 8 + 8
    RLAST = n - (NS - 1) * RA
    assert epw % C == 0 and f % LANES == 0 and RLAST > 0 and C % LANES == 0
    assert nchunks % (2 * NG) == 0 and G % 8 == 0

    mesh = plsc.VectorSubcoreMesh(
        core_axis_name="c", subcore_axis_name="s",
        num_cores=NC, num_subcores=NS,
    )

    params = (pltpu.CompilerParams(use_tc_tiling_on_sc=False)
              if sc_tiling else None)

    @functools.partial(
        pl.kernel,
        out_type=jax.ShapeDtypeStruct((2 * n, f), jnp.float32),
        mesh=mesh,
        compiler_params=params,
        scratch_types=[
            pltpu.VMEM((G,), jnp.int32),
            pltpu.VMEM((G,), jnp.int32),
            pltpu.VMEM((G,), jnp.float32),
            pltpu.VMEM((C,), jnp.int32),
            pltpu.VMEM((C,), jnp.int32),
            pltpu.VMEM((C, f), jnp.float32),
            pltpu.VMEM((C, f), jnp.float32),
            pltpu.VMEM((C, f), jnp.float32),
            pltpu.VMEM((C, f), jnp.float32),
            pltpu.VMEM_SHARED((n, f), jnp.float32),
            pltpu.SemaphoreType.DMA,
            pltpu.SemaphoreType.DMA,
            pltpu.SemaphoreType.DMA,
            pltpu.SemaphoreType.DMA,
        ],
    )
    def spmm(*refs):
        if half:
            (featx, featy, src_hbm, dst_hbm, w_hbm, z_hbm, out_hbm,
             src_g, dst_g, w_g, dst_v0, dst_v1, rows0, rows1, sc0, sc1,
             acc_sh, g0, g1, s0, s1) = refs
        else:
            (featx, src_hbm, dst_hbm, w_hbm, z_hbm, out_hbm,
             src_g, dst_g, w_g, dst_v0, dst_v1, rows0, rows1, sc0, sc1,
             acc_sh, g0, g1, s0, s1) = refs
            featy = featx
        c = lax.axis_index("c")
        s = lax.axis_index("s")

        def acc_io(copy):
            @pl.when(s < NS - 1)
            def _():
                copy(pl.multiple_of(s * RA, 8), RA)

            @pl.when(s == NS - 1)
            def _():
                copy((NS - 1) * RA, RLAST)

        acc_io(lambda r0, nr: pltpu.sync_copy(
            z_hbm.at[pl.ds(r0, nr)], acc_sh.at[pl.ds(r0, nr)]))
        plsc.subcore_barrier()

        base = s * epw if half else (c * NS + s) * epw
        row_off = c * n

        def issue_gather(kin, rows_buf, gsem):
            idxs = src_g.at[pl.ds(kin * C, C)]
            if half:
                @pl.when(c == 0)
                def _():
                    pltpu.async_copy(featx.at[idxs], rows_buf, gsem)

                @pl.when(c == 1)
                def _():
                    pltpu.async_copy(featy.at[idxs], rows_buf, gsem)
            else:
                pltpu.async_copy(featx.at[idxs], rows_buf, gsem)

        def process(kp, kin, rows_buf, gsem, sc_buf, dst_vp, ssem):
            pltpu.make_async_copy(
                featx.at[src_g.at[pl.ds(0, C)]], rows_buf, gsem).wait()
            @pl.when(kp > 0)
            def _():
                pltpu.make_async_copy(
                    sc_buf, acc_sh.at[dst_vp], ssem).wait()
            for g in range(C // LANES):
                wvec = w_g[pl.ds(kin * C + g * LANES, LANES)]
                for t in range(LANES):
                    ei = g * LANES + t
                    wsplat = jnp.full((LANES,), wvec[t], jnp.float32)
                    for fj in range(f // LANES):
                        fsl = pl.ds(fj * LANES, LANES)
                        sc_buf[ei, fsl] = rows_buf[ei, fsl] * wsplat
            for j in range(C // LANES):
                sl = pl.ds(j * LANES, LANES)
                dst_vp[sl] = dst_g[pl.ds(kin * C + j * LANES, LANES)]
            pltpu.async_copy(sc_buf, acc_sh.at[dst_vp], ssem, add=True)

        def pair_body(kp, carry):
            @pl.when(kp % ppg == 0)
            def _():
                goff = base + (kp // ppg) * G
                pltpu.sync_copy(src_hbm.at[pl.ds(goff, G)], src_g)
                pltpu.sync_copy(dst_hbm.at[pl.ds(goff, G)], dst_g)
                pltpu.sync_copy(w_hbm.at[pl.ds(goff, G)], w_g)
                issue_gather(0, rows0, g0)

            kin = (kp % ppg) * 2
            issue_gather(kin + 1, rows1, g1)
            process(kp, kin, rows0, g0, sc0, dst_v0, s0)

            @pl.when(kin + 2 < GC)
            def _():
                issue_gather(kin + 2, rows0, g0)

            process(kp, kin + 1, rows1, g1, sc1, dst_v1, s1)
            return carry

        lax.fori_loop(0, npairs, pair_body, 0)
        pltpu.make_async_copy(sc0, acc_sh.at[dst_v0], s0).wait()
        pltpu.make_async_copy(sc1, acc_sh.at[dst_v1], s1).wait()

        plsc.subcore_barrier()
        acc_io(lambda r0, nr: pltpu.sync_copy(
            acc_sh.at[pl.ds(r0, nr)],
            out_hbm.at[pl.ds(pl.multiple_of(row_off + r0, 8), nr)]))

    return spmm


EP = 327680

_spmm_h = _make_spmm(N, EP, NHID, split="half")
_spmm_c = _make_spmm(N, EP, NCLASS, split="half", sc_tiling=True)


def kernel(x, y, edge_index, edge_weight, W1, b1, W2, b2):
    pad_idx = jnp.arange(EP - E, dtype=jnp.int32) % N
    dst = jnp.concatenate([edge_index[0], pad_idx])
    src = jnp.concatenate([edge_index[1], pad_idx])
    w = jnp.concatenate([edge_weight, jnp.zeros((EP - E,), jnp.float32)])
    zeros_h = jnp.zeros((N, NHID), jnp.float32)
    zeros_c = jnp.zeros((N, NCLASS), jnp.float32)

    sx, sy = _mm_dual(x, y, W1)
    t = _spmm_h(sx, sy, src, dst, w, zeros_h)
    h2x, h2y = _stage_b(t, b1, W2)
    u = _spmm_c(h2x, h2y, src, dst, w, zeros_c)
    return _stage_c(u, b2)


# speedup vs baseline: 1.1258x; 1.1258x over previous
import functools

import jax
import jax.numpy as jnp
from jax import lax
from jax.experimental import pallas as pl
from jax.experimental.pallas import tpu as pltpu
from jax.experimental.pallas import tpu_sc as plsc

N = 10000
E = 320000
NFEAT = 128
NHID = 128
NCLASS = 64

NC = 2
NS = 16
LANES = 16

BR = 400


def _mm_dual_body(x_ref, y_ref, w_ref, ox_ref, oy_ref):
    ox_ref[...] = jnp.dot(x_ref[...], w_ref[...], preferred_element_type=jnp.float32)
    oy_ref[...] = jnp.dot(y_ref[...], w_ref[...], preferred_element_type=jnp.float32)


def _mm_dual(x, y, W1):
    n, k = x.shape
    h = W1.shape[1]
    return pl.pallas_call(
        _mm_dual_body,
        grid=(n // BR,),
        in_specs=[
            pl.BlockSpec((BR, k), lambda i: (i, 0)),
            pl.BlockSpec((BR, k), lambda i: (i, 0)),
            pl.BlockSpec((k, h), lambda i: (0, 0)),
        ],
        out_specs=[
            pl.BlockSpec((BR, h), lambda i: (i, 0)),
            pl.BlockSpec((BR, h), lambda i: (i, 0)),
        ],
        out_shape=[
            jax.ShapeDtypeStruct((n, h), jnp.float32),
            jax.ShapeDtypeStruct((n, h), jnp.float32),
        ],
    )(x, y, W1)


def _stage_b_body(tx_ref, ty_ref, b1_ref, w2_ref, o_ref):
    hx = jnp.maximum(tx_ref[...] + b1_ref[...], 0.0)
    hy = jnp.maximum(ty_ref[...] + b1_ref[...], 0.0)
    ox = jnp.dot(hx, w2_ref[...], preferred_element_type=jnp.float32)
    oy = jnp.dot(hy, w2_ref[...], preferred_element_type=jnp.float32)
    o_ref[...] = jnp.concatenate([ox, oy], axis=1)


def _stage_b(t, b1, W2):
    n2, h = t.shape
    n = n2 // 2
    c = W2.shape[1]
    nb = n // BR
    return pl.pallas_call(
        _stage_b_body,
        grid=(nb,),
        in_specs=[
            pl.BlockSpec((BR, h), lambda i: (i, 0)),
            pl.BlockSpec((BR, h), lambda i: (i + nb, 0)),
            pl.BlockSpec((1, h), lambda i: (0, 0)),
            pl.BlockSpec((h, c), lambda i: (0, 0)),
        ],
        out_specs=pl.BlockSpec((BR, 2 * c), lambda i: (i, 0)),
        out_shape=jax.ShapeDtypeStruct((n, 2 * c), jnp.float32),
    )(t, t, b1.reshape(1, h), W2)


def _stage_c_body(px_ref, py_ref, b2_ref, o_ref):
    z = px_ref[...] + py_ref[...] + b2_ref[...]
    m = jnp.max(z, axis=1, keepdims=True)
    ez = jnp.exp(z - m)
    lse = jnp.log(jnp.sum(ez, axis=1, keepdims=True)) + m
    o_ref[...] = z - lse


def _stage_c(p, b2t):
    n2, w = p.shape
    n = n2 // 2
    nb = n // BR
    return pl.pallas_call(
        _stage_c_body,
        grid=(nb,),
        in_specs=[
            pl.BlockSpec((BR, w), lambda i: (i, 0)),
            pl.BlockSpec((BR, w), lambda i: (i + nb, 0)),
            pl.BlockSpec((1, w), lambda i: (0, 0)),
        ],
        out_specs=pl.BlockSpec((BR, w), lambda i: (i, 0)),
        out_shape=jax.ShapeDtypeStruct((n, w), jnp.float32),
    )(p, p, b2t.reshape(1, w))


def _make_spmm(n, e, f, split):
    half = split == "half"
    C = 64
    epw = e // NS if half else e // (NC * NS)
    nchunks = epw // C
    npairs = nchunks // 2
    NG = 5
    GC = nchunks // NG
    G = GC * C
    ppg = GC // 2
    RA = -(-n // NS) // 8 * 8 + 8
    RLAST = n - (NS - 1) * RA
    assert epw % C == 0 and f % LANES == 0 and RLAST > 0 and C % LANES == 0
    assert nchunks % (2 * NG) == 0 and G % 8 == 0

    mesh = plsc.VectorSubcoreMesh(
        core_axis_name="c", subcore_axis_name="s",
        num_cores=NC, num_subcores=NS,
    )

    @functools.partial(
        pl.kernel,
        out_type=jax.ShapeDtypeStruct((2 * n, f), jnp.float32),
        mesh=mesh,
        scratch_types=[
            pltpu.VMEM((G,), jnp.int32),
            pltpu.VMEM((G,), jnp.int32),
            pltpu.VMEM((G,), jnp.float32),
            pltpu.VMEM((C,), jnp.int32),
            pltpu.VMEM((C,), jnp.int32),
            pltpu.VMEM((C, f), jnp.float32),
            pltpu.VMEM((C, f), jnp.float32),
            pltpu.VMEM((C, f), jnp.float32),
            pltpu.VMEM((C, f), jnp.float32),
            pltpu.VMEM_SHARED((n, f), jnp.float32),
            pltpu.SemaphoreType.DMA,
            pltpu.SemaphoreType.DMA,
            pltpu.SemaphoreType.DMA,
            pltpu.SemaphoreType.DMA,
        ],
    )
    def spmm(*refs):
        if half:
            (featx, featy, src_hbm, dst_hbm, w_hbm, out_hbm,
             src_g, dst_g, w_g, dst_v0, dst_v1, rows0, rows1, sc0, sc1,
             acc_sh, g0, g1, s0, s1) = refs
        else:
            (featx, src_hbm, dst_hbm, w_hbm, out_hbm,
             src_g, dst_g, w_g, dst_v0, dst_v1, rows0, rows1, sc0, sc1,
             acc_sh, g0, g1, s0, s1) = refs
            featy = featx
        c = lax.axis_index("c")
        s = lax.axis_index("s")

        def acc_io(copy):
            @pl.when(s < NS - 1)
            def _():
                copy(pl.multiple_of(s * RA, 8), RA)

            @pl.when(s == NS - 1)
            def _():
                copy((NS - 1) * RA, RLAST)

        zv = jnp.zeros((LANES,), jnp.float32)
        for r in range(C):
            for fj in range(f // LANES):
                sc0[r, pl.ds(fj * LANES, LANES)] = zv

        def zero_rows(r0, nr):
            for off in range(0, nr, C):
                sz = min(C, nr - off)
                pltpu.sync_copy(sc0.at[pl.ds(0, sz)],
                                acc_sh.at[pl.ds(r0 + off, sz)])

        acc_io(zero_rows)
        plsc.subcore_barrier()

        base = s * epw if half else (c * NS + s) * epw
        row_off = c * n

        def issue_gather(kin, rows_buf, gsem):
            idxs = src_g.at[pl.ds(kin * C, C)]
            if half:
                @pl.when(c == 0)
                def _():
                    pltpu.async_copy(featx.at[idxs], rows_buf, gsem)

                @pl.when(c == 1)
                def _():
                    pltpu.async_copy(featy.at[idxs], rows_buf, gsem)
            else:
                pltpu.async_copy(featx.at[idxs], rows_buf, gsem)

        def process(kp, kin, rows_buf, gsem, sc_buf, dst_vp, ssem):
            pltpu.make_async_copy(
                featx.at[src_g.at[pl.ds(0, C)]], rows_buf, gsem).wait()
            @pl.when(kp > 0)
            def _():
                pltpu.make_async_copy(
                    sc_buf, acc_sh.at[dst_vp], ssem).wait()
            for g in range(C // LANES):
                wvec = w_g[pl.ds(kin * C + g * LANES, LANES)]
                for t in range(LANES):
                    ei = g * LANES + t
                    wsplat = jnp.full((LANES,), wvec[t], jnp.float32)
                    for fj in range(f // LANES):
                        fsl = pl.ds(fj * LANES, LANES)
                        sc_buf[ei, fsl] = rows_buf[ei, fsl] * wsplat
            for j in range(C // LANES):
                sl = pl.ds(j * LANES, LANES)
                dst_vp[sl] = dst_g[pl.ds(kin * C + j * LANES, LANES)]
            pltpu.async_copy(sc_buf, acc_sh.at[dst_vp], ssem, add=True)

        def pair_body(kp, carry):
            @pl.when(kp % ppg == 0)
            def _():
                goff = base + (kp // ppg) * G
                pltpu.sync_copy(src_hbm.at[pl.ds(goff, G)], src_g)
                pltpu.sync_copy(dst_hbm.at[pl.ds(goff, G)], dst_g)
                pltpu.sync_copy(w_hbm.at[pl.ds(goff, G)], w_g)
                issue_gather(0, rows0, g0)

            kin = (kp % ppg) * 2
            issue_gather(kin + 1, rows1, g1)
            process(kp, kin, rows0, g0, sc0, dst_v0, s0)

            @pl.when(kin + 2 < GC)
            def _():
                issue_gather(kin + 2, rows0, g0)

            process(kp, kin + 1, rows1, g1, sc1, dst_v1, s1)
            return carry

        lax.fori_loop(0, npairs, pair_body, 0)
        pltpu.make_async_copy(sc0, acc_sh.at[dst_v0], s0).wait()
        pltpu.make_async_copy(sc1, acc_sh.at[dst_v1], s1).wait()

        plsc.subcore_barrier()
        acc_io(lambda r0, nr: pltpu.sync_copy(
            acc_sh.at[pl.ds(r0, nr)],
            out_hbm.at[pl.ds(pl.multiple_of(row_off + r0, 8), nr)]))

    return spmm


EP = 327680

_spmm_h = _make_spmm(N, EP, NHID, split="half")
_spmm_c = _make_spmm(N, EP, 2 * NCLASS, split="edges")


def kernel(x, y, edge_index, edge_weight, W1, b1, W2, b2):
    pad_idx = jnp.arange(EP - E, dtype=jnp.int32) % N
    dst = jnp.concatenate([edge_index[0], pad_idx])
    src = jnp.concatenate([edge_index[1], pad_idx])
    w = jnp.concatenate([edge_weight, jnp.zeros((EP - E,), jnp.float32)])
    b2t = jnp.concatenate([b2, b2])

    sx, sy = _mm_dual(x, y, W1)
    t = _spmm_h(sx, sy, src, dst, w)
    h = _stage_b(t, b1, W2)
    p = _spmm_c(h, src, dst, w)
    return _stage_c(p, b2t)


# speedup vs baseline: 1.2549x; 1.1147x over previous
import functools

import jax
import jax.numpy as jnp
from jax import lax
from jax.experimental import pallas as pl
from jax.experimental.pallas import tpu as pltpu
from jax.experimental.pallas import tpu_sc as plsc

N = 10000
E = 320000
NFEAT = 128
NHID = 128
NCLASS = 64

NC = 2
NS = 16
LANES = 16

BR = 2000


def _mm_dual_body(x_ref, y_ref, w_ref, ox_ref, oy_ref):
    ox_ref[...] = jnp.dot(x_ref[...], w_ref[...], preferred_element_type=jnp.float32)
    oy_ref[...] = jnp.dot(y_ref[...], w_ref[...], preferred_element_type=jnp.float32)


def _mm_dual(x, y, W1):
    n, k = x.shape
    h = W1.shape[1]
    return pl.pallas_call(
        _mm_dual_body,
        grid=(n // BR,),
        in_specs=[
            pl.BlockSpec((BR, k), lambda i: (i, 0)),
            pl.BlockSpec((BR, k), lambda i: (i, 0)),
            pl.BlockSpec((k, h), lambda i: (0, 0)),
        ],
        out_specs=[
            pl.BlockSpec((BR, h), lambda i: (i, 0)),
            pl.BlockSpec((BR, h), lambda i: (i, 0)),
        ],
        out_shape=[
            jax.ShapeDtypeStruct((n, h), jnp.float32),
            jax.ShapeDtypeStruct((n, h), jnp.float32),
        ],
    )(x, y, W1)


def _stage_b_body(tx_ref, ty_ref, b1_ref, w2_ref, o_ref):
    hx = jnp.maximum(tx_ref[...] + b1_ref[...], 0.0)
    hy = jnp.maximum(ty_ref[...] + b1_ref[...], 0.0)
    ox = jnp.dot(hx, w2_ref[...], preferred_element_type=jnp.float32)
    oy = jnp.dot(hy, w2_ref[...], preferred_element_type=jnp.float32)
    o_ref[...] = jnp.concatenate([ox, oy], axis=1)


def _stage_b(t, b1, W2):
    n2, h = t.shape
    n = n2 // 2
    c = W2.shape[1]
    nb = n // BR
    return pl.pallas_call(
        _stage_b_body,
        grid=(nb,),
        in_specs=[
            pl.BlockSpec((BR, h), lambda i: (i, 0)),
            pl.BlockSpec((BR, h), lambda i: (i + nb, 0)),
            pl.BlockSpec((1, h), lambda i: (0, 0)),
            pl.BlockSpec((h, c), lambda i: (0, 0)),
        ],
        out_specs=pl.BlockSpec((BR, 2 * c), lambda i: (i, 0)),
        out_shape=jax.ShapeDtypeStruct((n, 2 * c), jnp.float32),
    )(t, t, b1.reshape(1, h), W2)


def _stage_c_body(px_ref, py_ref, b2_ref, o_ref):
    z = px_ref[...] + py_ref[...] + b2_ref[...]
    m = jnp.max(z, axis=1, keepdims=True)
    ez = jnp.exp(z - m)
    lse = jnp.log(jnp.sum(ez, axis=1, keepdims=True)) + m
    o_ref[...] = z - lse


def _stage_c(p, b2t):
    n2, w = p.shape
    n = n2 // 2
    nb = n // BR
    return pl.pallas_call(
        _stage_c_body,
        grid=(nb,),
        in_specs=[
            pl.BlockSpec((BR, w), lambda i: (i, 0)),
            pl.BlockSpec((BR, w), lambda i: (i + nb, 0)),
            pl.BlockSpec((1, w), lambda i: (0, 0)),
        ],
        out_specs=pl.BlockSpec((BR, w), lambda i: (i, 0)),
        out_shape=jax.ShapeDtypeStruct((n, w), jnp.float32),
    )(p, p, b2t.reshape(1, w))


def _make_spmm(n, e, f, split):
    half = split == "half"
    C = 64
    epw = e // NS if half else e // (NC * NS)
    nchunks = epw // C
    npairs = nchunks // 2
    NG = 10
    GC = nchunks // NG
    G = GC * C
    ppg = GC // 2
    RA = -(-n // NS) // 8 * 8 + 8
    RLAST = n - (NS - 1) * RA
    assert epw % C == 0 and f % LANES == 0 and RLAST > 0 and C % LANES == 0
    assert nchunks % (2 * NG) == 0 and G % 8 == 0

    mesh = plsc.VectorSubcoreMesh(
        core_axis_name="c", subcore_axis_name="s",
        num_cores=NC, num_subcores=NS,
    )

    @functools.partial(
        pl.kernel,
        out_type=jax.ShapeDtypeStruct((2 * n, f), jnp.float32),
        mesh=mesh,
        scratch_types=[
            pltpu.VMEM((2 * G,), jnp.int32),
            pltpu.VMEM((2 * G,), jnp.int32),
            pltpu.VMEM((2 * G,), jnp.float32),
            pltpu.VMEM((C,), jnp.int32),
            pltpu.VMEM((C,), jnp.int32),
            pltpu.VMEM((C, f), jnp.float32),
            pltpu.VMEM((C, f), jnp.float32),
            pltpu.VMEM((C, f), jnp.float32),
            pltpu.VMEM((C, f), jnp.float32),
            pltpu.VMEM_SHARED((n, f), jnp.float32),
            pltpu.SemaphoreType.DMA,
            pltpu.SemaphoreType.DMA,
            pltpu.SemaphoreType.DMA,
            pltpu.SemaphoreType.DMA,
            pltpu.SemaphoreType.DMA,
        ],
    )
    def spmm(*refs):
        if half:
            (featx, featy, src_hbm, dst_hbm, w_hbm, out_hbm,
             src_g, dst_g, w_g, dst_v0, dst_v1, rows0, rows1, sc0, sc1,
             acc_sh, isem, g0, g1, s0, s1) = refs
        else:
            (featx, src_hbm, dst_hbm, w_hbm, out_hbm,
             src_g, dst_g, w_g, dst_v0, dst_v1, rows0, rows1, sc0, sc1,
             acc_sh, isem, g0, g1, s0, s1) = refs
            featy = featx
        c = lax.axis_index("c")
        s = lax.axis_index("s")

        def acc_io(copy):
            @pl.when(s < NS - 1)
            def _():
                copy(pl.multiple_of(s * RA, 8), RA)

            @pl.when(s == NS - 1)
            def _():
                copy((NS - 1) * RA, RLAST)

        zv = jnp.zeros((LANES,), jnp.float32)
        for r in range(C):
            for fj in range(f // LANES):
                sc0[r, pl.ds(fj * LANES, LANES)] = zv

        def zero_rows(r0, nr):
            for off in range(0, nr, C):
                sz = min(C, nr - off)
                pltpu.sync_copy(sc0.at[pl.ds(0, sz)],
                                acc_sh.at[pl.ds(r0 + off, sz)])

        acc_io(zero_rows)
        plsc.subcore_barrier()

        base = s * epw if half else (c * NS + s) * epw
        row_off = c * n

        def issue_idx_load(g, hofs):
            goff = base + g * G
            pltpu.async_copy(src_hbm.at[pl.ds(goff, G)],
                             src_g.at[pl.ds(hofs, G)], isem)
            pltpu.async_copy(dst_hbm.at[pl.ds(goff, G)],
                             dst_g.at[pl.ds(hofs, G)], isem)
            pltpu.async_copy(w_hbm.at[pl.ds(goff, G)],
                             w_g.at[pl.ds(hofs, G)], isem)

        def wait_idx_load(hofs):
            pltpu.make_async_copy(src_hbm.at[pl.ds(base, G)],
                                  src_g.at[pl.ds(hofs, G)], isem).wait()
            pltpu.make_async_copy(dst_hbm.at[pl.ds(base, G)],
                                  dst_g.at[pl.ds(hofs, G)], isem).wait()
            pltpu.make_async_copy(w_hbm.at[pl.ds(base, G)],
                                  w_g.at[pl.ds(hofs, G)], isem).wait()

        def issue_gather(bofs, kin, rows_buf, gsem):
            idxs = src_g.at[pl.ds(bofs + kin * C, C)]
            if half:
                @pl.when(c == 0)
                def _():
                    pltpu.async_copy(featx.at[idxs], rows_buf, gsem)

                @pl.when(c == 1)
                def _():
                    pltpu.async_copy(featy.at[idxs], rows_buf, gsem)
            else:
                pltpu.async_copy(featx.at[idxs], rows_buf, gsem)

        def process(kp, bofs, kin, rows_buf, gsem, sc_buf, dst_vp, ssem):
            pltpu.make_async_copy(
                featx.at[src_g.at[pl.ds(0, C)]], rows_buf, gsem).wait()
            @pl.when(kp > 0)
            def _():
                pltpu.make_async_copy(
                    sc_buf, acc_sh.at[dst_vp], ssem).wait()
            for g in range(C // LANES):
                wvec = w_g[pl.ds(bofs + kin * C + g * LANES, LANES)]
                for t in range(LANES):
                    ei = g * LANES + t
                    wsplat = jnp.full((LANES,), wvec[t], jnp.float32)
                    for fj in range(f // LANES):
                        fsl = pl.ds(fj * LANES, LANES)
                        sc_buf[ei, fsl] = rows_buf[ei, fsl] * wsplat
            for j in range(C // LANES):
                sl = pl.ds(j * LANES, LANES)
                dst_vp[sl] = dst_g[pl.ds(bofs + kin * C + j * LANES, LANES)]
            pltpu.async_copy(sc_buf, acc_sh.at[dst_vp], ssem, add=True)

        def pair_body(kp, carry):
            g = kp // ppg
            hb = lax.rem(g, 2)
            bofs = hb * G
            nofs = (1 - hb) * G
            kin = (kp % ppg) * 2

            @pl.when((kp % ppg == 0) & (g + 1 < NG))
            def _():
                issue_idx_load(g + 1, nofs)

            issue_gather(bofs, kin + 1, rows1, g1)
            process(kp, bofs, kin, rows0, g0, sc0, dst_v0, s0)

            @pl.when(kin + 2 < GC)
            def _():
                issue_gather(bofs, kin + 2, rows0, g0)

            @pl.when((kin + 2 >= GC) & (kp + 1 < npairs))
            def _():
                wait_idx_load(nofs)
                issue_gather(nofs, 0, rows0, g0)

            process(kp, bofs, kin + 1, rows1, g1, sc1, dst_v1, s1)
            return carry

        issue_idx_load(0, 0)
        wait_idx_load(0)
        issue_gather(0, 0, rows0, g0)
        lax.fori_loop(0, npairs, pair_body, 0)
        pltpu.make_async_copy(sc0, acc_sh.at[dst_v0], s0).wait()
        pltpu.make_async_copy(sc1, acc_sh.at[dst_v1], s1).wait()

        plsc.subcore_barrier()
        acc_io(lambda r0, nr: pltpu.sync_copy(
            acc_sh.at[pl.ds(r0, nr)],
            out_hbm.at[pl.ds(pl.multiple_of(row_off + r0, 8), nr)]))

    return spmm


EP = 327680

_spmm_h = _make_spmm(N, EP, NHID, split="half")
_spmm_c = _make_spmm(N, EP, 2 * NCLASS, split="edges")


def kernel(x, y, edge_index, edge_weight, W1, b1, W2, b2):
    pad_idx = jnp.arange(EP - E, dtype=jnp.int32) % N
    dst = jnp.concatenate([edge_index[0], pad_idx])
    src = jnp.concatenate([edge_index[1], pad_idx])
    w = jnp.concatenate([edge_weight, jnp.zeros((EP - E,), jnp.float32)])
    b2t = jnp.concatenate([b2, b2])

    sx, sy = _mm_dual(x, y, W1)
    t = _spmm_h(sx, sy, src, dst, w)
    h = _stage_b(t, b1, W2)
    p = _spmm_c(h, src, dst, w)
    return _stage_c(p, b2t)


# speedup vs baseline: 1.2551x; 1.0002x over previous
import functools

import jax
import jax.numpy as jnp
from jax import lax
from jax.experimental import pallas as pl
from jax.experimental.pallas import tpu as pltpu
from jax.experimental.pallas import tpu_sc as plsc

N = 10000
E = 320000
NFEAT = 128
NHID = 128
NCLASS = 64

NC = 2
NS = 16
LANES = 16

BR = 2000


def _mm_dual_body(x_ref, y_ref, w_ref, ox_ref, oy_ref):
    ox_ref[...] = jnp.dot(x_ref[...], w_ref[...], preferred_element_type=jnp.float32)
    oy_ref[...] = jnp.dot(y_ref[...], w_ref[...], preferred_element_type=jnp.float32)


def _mm_dual(x, y, W1):
    n, k = x.shape
    h = W1.shape[1]
    return pl.pallas_call(
        _mm_dual_body,
        grid=(n // BR,),
        in_specs=[
            pl.BlockSpec((BR, k), lambda i: (i, 0)),
            pl.BlockSpec((BR, k), lambda i: (i, 0)),
            pl.BlockSpec((k, h), lambda i: (0, 0)),
        ],
        out_specs=[
            pl.BlockSpec((BR, h), lambda i: (i, 0)),
            pl.BlockSpec((BR, h), lambda i: (i, 0)),
        ],
        out_shape=[
            jax.ShapeDtypeStruct((n, h), jnp.float32),
            jax.ShapeDtypeStruct((n, h), jnp.float32),
        ],
    )(x, y, W1)


def _stage_b_body(tx_ref, ty_ref, b1_ref, w2_ref, o_ref):
    hx = jnp.maximum(tx_ref[...] + b1_ref[...], 0.0)
    hy = jnp.maximum(ty_ref[...] + b1_ref[...], 0.0)
    ox = jnp.dot(hx, w2_ref[...], preferred_element_type=jnp.float32)
    oy = jnp.dot(hy, w2_ref[...], preferred_element_type=jnp.float32)
    o_ref[...] = jnp.concatenate([ox, oy], axis=1)


def _stage_b(t, b1, W2):
    n2, h = t.shape
    n = n2 // 2
    c = W2.shape[1]
    nb = n // BR
    return pl.pallas_call(
        _stage_b_body,
        grid=(nb,),
        in_specs=[
            pl.BlockSpec((BR, h), lambda i: (i, 0)),
            pl.BlockSpec((BR, h), lambda i: (i + nb, 0)),
            pl.BlockSpec((1, h), lambda i: (0, 0)),
            pl.BlockSpec((h, c), lambda i: (0, 0)),
        ],
        out_specs=pl.BlockSpec((BR, 2 * c), lambda i: (i, 0)),
        out_shape=jax.ShapeDtypeStruct((n, 2 * c), jnp.float32),
    )(t, t, b1.reshape(1, h), W2)


def _stage_c_body(px_ref, py_ref, b2_ref, o_ref):
    z = px_ref[...] + py_ref[...] + b2_ref[...]
    m = jnp.max(z, axis=1, keepdims=True)
    ez = jnp.exp(z - m)
    lse = jnp.log(jnp.sum(ez, axis=1, keepdims=True)) + m
    o_ref[...] = z - lse


def _stage_c(p, b2t):
    n2, w = p.shape
    n = n2 // 2
    nb = n // BR
    return pl.pallas_call(
        _stage_c_body,
        grid=(nb,),
        in_specs=[
            pl.BlockSpec((BR, w), lambda i: (i, 0)),
            pl.BlockSpec((BR, w), lambda i: (i + nb, 0)),
            pl.BlockSpec((1, w), lambda i: (0, 0)),
        ],
        out_specs=pl.BlockSpec((BR, w), lambda i: (i, 0)),
        out_shape=jax.ShapeDtypeStruct((n, w), jnp.float32),
    )(p, p, b2t.reshape(1, w))


def _make_spmm(n, e, f, split):
    half = split == "half"
    C = 64
    epw = e // NS if half else e // (NC * NS)
    nchunks = epw // C
    npairs = nchunks // 2
    NG = 10
    GC = nchunks // NG
    G = GC * C
    ppg = GC // 2
    RA = -(-n // NS) // 8 * 8 + 8
    RLAST = n - (NS - 1) * RA
    assert epw % C == 0 and f % LANES == 0 and RLAST > 0 and C % LANES == 0
    assert nchunks % (2 * NG) == 0 and G % 8 == 0

    mesh = plsc.VectorSubcoreMesh(
        core_axis_name="c", subcore_axis_name="s",
        num_cores=NC, num_subcores=NS,
    )

    @functools.partial(
        pl.kernel,
        out_type=jax.ShapeDtypeStruct((2 * n, f), jnp.float32),
        mesh=mesh,
        scratch_types=[
            pltpu.VMEM((2 * G,), jnp.int32),
            pltpu.VMEM((2 * G,), jnp.int32),
            pltpu.VMEM((2 * G,), jnp.float32),
            pltpu.VMEM((C,), jnp.int32),
            pltpu.VMEM((C,), jnp.int32),
            pltpu.VMEM((C, f), jnp.float32),
            pltpu.VMEM((C, f), jnp.float32),
            pltpu.VMEM((C, f), jnp.float32),
            pltpu.VMEM((C, f), jnp.float32),
            pltpu.VMEM_SHARED((n, f), jnp.float32),
            pltpu.SemaphoreType.DMA,
            pltpu.SemaphoreType.DMA,
            pltpu.SemaphoreType.DMA,
            pltpu.SemaphoreType.DMA,
            pltpu.SemaphoreType.DMA,
        ],
    )
    def spmm(*refs):
        if half:
            (featx, featy, src_hbm, dst_hbm, w_hbm, out_hbm,
             src_g, dst_g, w_g, dst_v0, dst_v1, rows0, rows1, sc0, sc1,
             acc_sh, isem, g0, g1, s0, s1) = refs
        else:
            (featx, src_hbm, dst_hbm, w_hbm, out_hbm,
             src_g, dst_g, w_g, dst_v0, dst_v1, rows0, rows1, sc0, sc1,
             acc_sh, isem, g0, g1, s0, s1) = refs
            featy = featx
        c = lax.axis_index("c")
        s = lax.axis_index("s")

        def acc_io(copy):
            @pl.when(s < NS - 1)
            def _():
                copy(pl.multiple_of(s * RA, 8), RA)

            @pl.when(s == NS - 1)
            def _():
                copy((NS - 1) * RA, RLAST)

        zv = jnp.zeros((LANES,), jnp.float32)
        for r in range(C):
            for fj in range(f // LANES):
                sc0[r, pl.ds(fj * LANES, LANES)] = zv

        def zero_rows(r0, nr):
            for off in range(0, nr, C):
                sz = min(C, nr - off)
                pltpu.sync_copy(sc0.at[pl.ds(0, sz)],
                                acc_sh.at[pl.ds(r0 + off, sz)])

        acc_io(zero_rows)
        plsc.subcore_barrier()

        base = s * epw if half else (c * NS + s) * epw
        row_off = c * n

        def issue_idx_load(g, hofs):
            goff = base + g * G
            pltpu.async_copy(src_hbm.at[pl.ds(goff, G)],
                             src_g.at[pl.ds(hofs, G)], isem)
            pltpu.async_copy(dst_hbm.at[pl.ds(goff, G)],
                             dst_g.at[pl.ds(hofs, G)], isem)
            pltpu.async_copy(w_hbm.at[pl.ds(goff, G)],
                             w_g.at[pl.ds(hofs, G)], isem)

        def wait_idx_load(hofs):
            pltpu.make_async_copy(src_hbm.at[pl.ds(base, G)],
                                  src_g.at[pl.ds(hofs, G)], isem).wait()
            pltpu.make_async_copy(dst_hbm.at[pl.ds(base, G)],
                                  dst_g.at[pl.ds(hofs, G)], isem).wait()
            pltpu.make_async_copy(w_hbm.at[pl.ds(base, G)],
                                  w_g.at[pl.ds(hofs, G)], isem).wait()

        def issue_gather(bofs, kin, rows_buf, gsem):
            H = C // 2
            for p in range(2):
                idxs = src_g.at[pl.ds(bofs + kin * C + p * H, H)]
                dst = rows_buf.at[pl.ds(p * H, H)]
                if half:
                    @pl.when(c == 0)
                    def _():
                        pltpu.async_copy(featx.at[idxs], dst, gsem)

                    @pl.when(c == 1)
                    def _():
                        pltpu.async_copy(featy.at[idxs], dst, gsem)
                else:
                    pltpu.async_copy(featx.at[idxs], dst, gsem)

        def process(kp, bofs, kin, rows_buf, gsem, sc_buf, dst_vp, ssem):
            pltpu.make_async_copy(
                featx.at[src_g.at[pl.ds(0, C)]], rows_buf, gsem).wait()
            @pl.when(kp > 0)
            def _():
                pltpu.make_async_copy(
                    sc_buf, acc_sh.at[dst_vp], ssem).wait()
            for g in range(C // LANES):
                wvec = w_g[pl.ds(bofs + kin * C + g * LANES, LANES)]
                for t in range(LANES):
                    ei = g * LANES + t
                    wsplat = jnp.full((LANES,), wvec[t], jnp.float32)
                    for fj in range(f // LANES):
                        fsl = pl.ds(fj * LANES, LANES)
                        sc_buf[ei, fsl] = rows_buf[ei, fsl] * wsplat
            for j in range(C // LANES):
                sl = pl.ds(j * LANES, LANES)
                dst_vp[sl] = dst_g[pl.ds(bofs + kin * C + j * LANES, LANES)]
            pltpu.async_copy(sc_buf, acc_sh.at[dst_vp], ssem, add=True)

        def pair_body(kp, carry):
            g = kp // ppg
            hb = lax.rem(g, 2)
            bofs = hb * G
            nofs = (1 - hb) * G
            kin = (kp % ppg) * 2

            @pl.when((kp % ppg == 0) & (g + 1 < NG))
            def _():
                issue_idx_load(g + 1, nofs)

            issue_gather(bofs, kin + 1, rows1, g1)
            process(kp, bofs, kin, rows0, g0, sc0, dst_v0, s0)

            @pl.when(kin + 2 < GC)
            def _():
                issue_gather(bofs, kin + 2, rows0, g0)

            @pl.when((kin + 2 >= GC) & (kp + 1 < npairs))
            def _():
                wait_idx_load(nofs)
                issue_gather(nofs, 0, rows0, g0)

            process(kp, bofs, kin + 1, rows1, g1, sc1, dst_v1, s1)
            return carry

        issue_idx_load(0, 0)
        wait_idx_load(0)
        issue_gather(0, 0, rows0, g0)
        lax.fori_loop(0, npairs, pair_body, 0)
        pltpu.make_async_copy(sc0, acc_sh.at[dst_v0], s0).wait()
        pltpu.make_async_copy(sc1, acc_sh.at[dst_v1], s1).wait()

        plsc.subcore_barrier()
        acc_io(lambda r0, nr: pltpu.sync_copy(
            acc_sh.at[pl.ds(r0, nr)],
            out_hbm.at[pl.ds(pl.multiple_of(row_off + r0, 8), nr)]))

    return spmm


EP = 327680

_spmm_h = _make_spmm(N, EP, NHID, split="half")
_spmm_c = _make_spmm(N, EP, 2 * NCLASS, split="edges")


def kernel(x, y, edge_index, edge_weight, W1, b1, W2, b2):
    pad_idx = jnp.arange(EP - E, dtype=jnp.int32) % N
    dst = jnp.concatenate([edge_index[0], pad_idx])
    src = jnp.concatenate([edge_index[1], pad_idx])
    w = jnp.concatenate([edge_weight, jnp.zeros((EP - E,), jnp.float32)])
    b2t = jnp.concatenate([b2, b2])

    sx, sy = _mm_dual(x, y, W1)
    t = _spmm_h(sx, sy, src, dst, w)
    h = _stage_b(t, b1, W2)
    p = _spmm_c(h, src, dst, w)
    return _stage_c(p, b2t)


# speedup vs baseline: 1.2614x; 1.0050x over previous
import functools

import jax
import jax.numpy as jnp
from jax import lax
from jax.experimental import pallas as pl
from jax.experimental.pallas import tpu as pltpu
from jax.experimental.pallas import tpu_sc as plsc

N = 10000
E = 320000
NFEAT = 128
NHID = 128
NCLASS = 64

NC = 2
NS = 16
LANES = 16

BR = 2000


def _mm_dual_body(x_ref, y_ref, w_ref, ox_ref, oy_ref):
    ox_ref[...] = jnp.dot(x_ref[...], w_ref[...], preferred_element_type=jnp.float32)
    oy_ref[...] = jnp.dot(y_ref[...], w_ref[...], preferred_element_type=jnp.float32)


def _mm_dual(x, y, W1):
    n, k = x.shape
    h = W1.shape[1]
    return pl.pallas_call(
        _mm_dual_body,
        grid=(n // BR,),
        in_specs=[
            pl.BlockSpec((BR, k), lambda i: (i, 0)),
            pl.BlockSpec((BR, k), lambda i: (i, 0)),
            pl.BlockSpec((k, h), lambda i: (0, 0)),
        ],
        out_specs=[
            pl.BlockSpec((BR, h), lambda i: (i, 0)),
            pl.BlockSpec((BR, h), lambda i: (i, 0)),
        ],
        out_shape=[
            jax.ShapeDtypeStruct((n, h), jnp.float32),
            jax.ShapeDtypeStruct((n, h), jnp.float32),
        ],
    )(x, y, W1)


def _stage_b_body(tx_ref, ty_ref, b1_ref, w2_ref, o_ref):
    hx = jnp.maximum(tx_ref[...] + b1_ref[...], 0.0)
    hy = jnp.maximum(ty_ref[...] + b1_ref[...], 0.0)
    ox = jnp.dot(hx, w2_ref[...], preferred_element_type=jnp.float32)
    oy = jnp.dot(hy, w2_ref[...], preferred_element_type=jnp.float32)
    o_ref[...] = jnp.concatenate([ox, oy], axis=1)


def _stage_b(t, b1, W2):
    n2, h = t.shape
    n = n2 // 2
    c = W2.shape[1]
    nb = n // BR
    return pl.pallas_call(
        _stage_b_body,
        grid=(nb,),
        in_specs=[
            pl.BlockSpec((BR, h), lambda i: (i, 0)),
            pl.BlockSpec((BR, h), lambda i: (i + nb, 0)),
            pl.BlockSpec((1, h), lambda i: (0, 0)),
            pl.BlockSpec((h, c), lambda i: (0, 0)),
        ],
        out_specs=pl.BlockSpec((BR, 2 * c), lambda i: (i, 0)),
        out_shape=jax.ShapeDtypeStruct((n, 2 * c), jnp.float32),
    )(t, t, b1.reshape(1, h), W2)


def _stage_c_body(px_ref, py_ref, b2_ref, o_ref):
    z = px_ref[...] + py_ref[...] + b2_ref[...]
    m = jnp.max(z, axis=1, keepdims=True)
    ez = jnp.exp(z - m)
    lse = jnp.log(jnp.sum(ez, axis=1, keepdims=True)) + m
    o_ref[...] = z - lse


def _stage_c(p, b2t):
    n2, w = p.shape
    n = n2 // 2
    nb = n // BR
    return pl.pallas_call(
        _stage_c_body,
        grid=(nb,),
        in_specs=[
            pl.BlockSpec((BR, w), lambda i: (i, 0)),
            pl.BlockSpec((BR, w), lambda i: (i + nb, 0)),
            pl.BlockSpec((1, w), lambda i: (0, 0)),
        ],
        out_specs=pl.BlockSpec((BR, w), lambda i: (i, 0)),
        out_shape=jax.ShapeDtypeStruct((n, w), jnp.float32),
    )(p, p, b2t.reshape(1, w))


def _make_spmm(n, e, f, split):
    half = split == "half"
    C = 64
    epw = e // NS if half else e // (NC * NS)
    nchunks = epw // C
    npairs = nchunks // 2
    NG = 10
    GC = nchunks // NG
    G = GC * C
    ppg = GC // 2
    RA = -(-n // NS) // 8 * 8 + 8
    RLAST = n - (NS - 1) * RA
    assert epw % C == 0 and f % LANES == 0 and RLAST > 0 and C % LANES == 0
    assert nchunks % (2 * NG) == 0 and G % 8 == 0

    mesh = plsc.VectorSubcoreMesh(
        core_axis_name="c", subcore_axis_name="s",
        num_cores=NC, num_subcores=NS,
    )

    @functools.partial(
        pl.kernel,
        out_type=jax.ShapeDtypeStruct((2 * n, f), jnp.float32),
        mesh=mesh,
        scratch_types=[
            pltpu.VMEM((2 * G,), jnp.int32),
            pltpu.VMEM((2 * G,), jnp.int32),
            pltpu.VMEM((2 * G,), jnp.float32),
            pltpu.VMEM((C,), jnp.int32),
            pltpu.VMEM((C,), jnp.int32),
            pltpu.VMEM((C, f), jnp.float32),
            pltpu.VMEM((C, f), jnp.float32),
            pltpu.VMEM((C, f), jnp.float32),
            pltpu.VMEM((C, f), jnp.float32),
            pltpu.VMEM_SHARED((n, f), jnp.float32),
            pltpu.SemaphoreType.DMA,
            pltpu.SemaphoreType.DMA,
            pltpu.SemaphoreType.DMA,
            pltpu.SemaphoreType.DMA,
            pltpu.SemaphoreType.DMA,
        ],
    )
    def spmm(*refs):
        if half:
            (featx, featy, src_hbm, dst_hbm, w_hbm, out_hbm,
             src_g, dst_g, w_g, dst_v0, dst_v1, rows0, rows1, sc0, sc1,
             acc_sh, isem, g0, g1, s0, s1) = refs
        else:
            (featx, src_hbm, dst_hbm, w_hbm, out_hbm,
             src_g, dst_g, w_g, dst_v0, dst_v1, rows0, rows1, sc0, sc1,
             acc_sh, isem, g0, g1, s0, s1) = refs
            featy = featx
        c = lax.axis_index("c")
        s = lax.axis_index("s")

        def acc_io(copy):
            @pl.when(s < NS - 1)
            def _():
                copy(pl.multiple_of(s * RA, 8), RA)

            @pl.when(s == NS - 1)
            def _():
                copy((NS - 1) * RA, RLAST)

        base = s * epw if half else (c * NS + s) * epw
        row_off = c * n

        def issue_idx_load(g, hofs):
            goff = base + g * G
            pltpu.async_copy(src_hbm.at[pl.ds(goff, G)],
                             src_g.at[pl.ds(hofs, G)], isem)
            pltpu.async_copy(dst_hbm.at[pl.ds(goff, G)],
                             dst_g.at[pl.ds(hofs, G)], isem)
            pltpu.async_copy(w_hbm.at[pl.ds(goff, G)],
                             w_g.at[pl.ds(hofs, G)], isem)

        def wait_idx_load(hofs):
            pltpu.make_async_copy(src_hbm.at[pl.ds(base, G)],
                                  src_g.at[pl.ds(hofs, G)], isem).wait()
            pltpu.make_async_copy(dst_hbm.at[pl.ds(base, G)],
                                  dst_g.at[pl.ds(hofs, G)], isem).wait()
            pltpu.make_async_copy(w_hbm.at[pl.ds(base, G)],
                                  w_g.at[pl.ds(hofs, G)], isem).wait()

        def issue_gather(bofs, kin, rows_buf, gsem):
            idxs = src_g.at[pl.ds(bofs + kin * C, C)]
            if half:
                @pl.when(c == 0)
                def _():
                    pltpu.async_copy(featx.at[idxs], rows_buf, gsem)

                @pl.when(c == 1)
                def _():
                    pltpu.async_copy(featy.at[idxs], rows_buf, gsem)
            else:
                pltpu.async_copy(featx.at[idxs], rows_buf, gsem)

        def process(kp, bofs, kin, rows_buf, gsem, sc_buf, dst_vp, ssem):
            pltpu.make_async_copy(
                featx.at[src_g.at[pl.ds(0, C)]], rows_buf, gsem).wait()
            @pl.when(kp > 0)
            def _():
                pltpu.make_async_copy(
                    sc_buf, acc_sh.at[dst_vp], ssem).wait()
            for g in range(C // LANES):
                wvec = w_g[pl.ds(bofs + kin * C + g * LANES, LANES)]
                for t in range(LANES):
                    ei = g * LANES + t
                    wsplat = jnp.full((LANES,), wvec[t], jnp.float32)
                    for fj in range(f // LANES):
                        fsl = pl.ds(fj * LANES, LANES)
                        sc_buf[ei, fsl] = rows_buf[ei, fsl] * wsplat
            for j in range(C // LANES):
                sl = pl.ds(j * LANES, LANES)
                dst_vp[sl] = dst_g[pl.ds(bofs + kin * C + j * LANES, LANES)]
            pltpu.async_copy(sc_buf, acc_sh.at[dst_vp], ssem, add=True)

        def pair_body(kp, carry):
            g = kp // ppg
            hb = lax.rem(g, 2)
            bofs = hb * G
            nofs = (1 - hb) * G
            kin = (kp % ppg) * 2

            @pl.when((kp % ppg == 0) & (g + 1 < NG))
            def _():
                issue_idx_load(g + 1, nofs)

            issue_gather(bofs, kin + 1, rows1, g1)
            process(kp, bofs, kin, rows0, g0, sc0, dst_v0, s0)

            @pl.when(kin + 2 < GC)
            def _():
                issue_gather(bofs, kin + 2, rows0, g0)

            @pl.when((kin + 2 >= GC) & (kp + 1 < npairs))
            def _():
                wait_idx_load(nofs)
                issue_gather(nofs, 0, rows0, g0)

            process(kp, bofs, kin + 1, rows1, g1, sc1, dst_v1, s1)
            return carry

        issue_idx_load(0, 0)
        zv = jnp.zeros((LANES,), jnp.float32)
        for r in range(C):
            for fj in range(f // LANES):
                sc0[r, pl.ds(fj * LANES, LANES)] = zv

        def zero_rows(r0, nr):
            for off in range(0, nr, C):
                sz = min(C, nr - off)
                pltpu.sync_copy(sc0.at[pl.ds(0, sz)],
                                acc_sh.at[pl.ds(r0 + off, sz)])

        acc_io(zero_rows)
        wait_idx_load(0)
        issue_gather(0, 0, rows0, g0)
        plsc.subcore_barrier()
        lax.fori_loop(0, npairs, pair_body, 0)
        pltpu.make_async_copy(sc0, acc_sh.at[dst_v0], s0).wait()
        pltpu.make_async_copy(sc1, acc_sh.at[dst_v1], s1).wait()

        plsc.subcore_barrier()
        acc_io(lambda r0, nr: pltpu.sync_copy(
            acc_sh.at[pl.ds(r0, nr)],
            out_hbm.at[pl.ds(pl.multiple_of(row_off + r0, 8), nr)]))

    return spmm


EP = 327680

_spmm_h = _make_spmm(N, EP, NHID, split="half")
_spmm_c = _make_spmm(N, EP, 2 * NCLASS, split="edges")


def kernel(x, y, edge_index, edge_weight, W1, b1, W2, b2):
    pad_idx = jnp.arange(EP - E, dtype=jnp.int32) % N
    dst = jnp.concatenate([edge_index[0], pad_idx])
    src = jnp.concatenate([edge_index[1], pad_idx])
    w = jnp.concatenate([edge_weight, jnp.zeros((EP - E,), jnp.float32)])
    b2t = jnp.concatenate([b2, b2])

    sx, sy = _mm_dual(x, y, W1)
    t = _spmm_h(sx, sy, src, dst, w)
    h = _stage_b(t, b1, W2)
    p = _spmm_c(h, src, dst, w)
    return _stage_c(p, b2t)
